# baseline (speedup 1.0000x reference)
import math
import functools

import jax
import jax.numpy as jnp
from jax import lax
from jax.experimental import pallas as pl
from jax.experimental.pallas import tpu as pltpu
from jax.experimental.pallas import tpu_sc as plsc

VOCAB = 1000000
EMBED = 64
D_MODEL = 128
B = 4096
L = 200

NC = 2
NS = 16
NW = NC * NS

N = B * L
R = N // NW
CH = 8
ROWS_PER_STREAM = 128
GROUP = CH * ROWS_PER_STREAM
NGROUP = R // GROUP


def _sc_gather(idx3, table):
    mesh = plsc.VectorSubcoreMesh(core_axis_name="c", subcore_axis_name="s")

    @functools.partial(
        pl.kernel,
        out_type=jax.ShapeDtypeStruct((N, EMBED), jnp.float32),
        mesh=mesh,
        scratch_types=[
            pltpu.VMEM((CH, ROWS_PER_STREAM), jnp.int32),
            pltpu.VMEM((GROUP, EMBED), jnp.float32),
            pltpu.SemaphoreType.DMA,
        ],
        compiler_params=pltpu.CompilerParams(use_tc_tiling_on_sc=False),
    )
    def k(idx_hbm, table_hbm, emb_hbm, idx_v, rows_v, sem):
        wid = lax.axis_index("s") * NC + lax.axis_index("c")
        base = wid * R

        def group_body(g, carry):
            pltpu.sync_copy(idx_hbm.at[wid, pl.ds(g * CH, CH)], idx_v)
            descs = [
                pltpu.async_copy(
                    table_hbm.at[idx_v.at[j]],
                    rows_v.at[pl.ds(j * ROWS_PER_STREAM, ROWS_PER_STREAM)],
                    sem,
                )
                for j in range(CH)
            ]
            for d in descs:
                d.wait()
            pltpu.sync_copy(rows_v, emb_hbm.at[pl.ds(base + g * GROUP, GROUP)])
            return carry

        lax.fori_loop(0, NGROUP, group_body, 0)

    return k(idx3, table)


BLK = 1024


def _tc_matmul(emb2, W2, b2):
    def body(emb_ref, w_ref, b_ref, out_ref):
        out_ref[...] = (
            jnp.dot(emb_ref[...], w_ref[...], preferred_element_type=jnp.float32)
            + b_ref[...]
        )

    return pl.pallas_call(
        body,
        grid=(N // 2 // BLK,),
        in_specs=[
            pl.BlockSpec((BLK, 2 * EMBED), lambda i: (i, 0)),
            pl.BlockSpec((2 * EMBED, 2 * D_MODEL), lambda i: (0, 0)),
            pl.BlockSpec((1, 2 * D_MODEL), lambda i: (0, 0)),
        ],
        out_specs=pl.BlockSpec((BLK, 2 * D_MODEL), lambda i: (i, 0)),
        out_shape=jax.ShapeDtypeStruct((N // 2, 2 * D_MODEL), jnp.float32),
    )(emb2, W2, b2)


def kernel(x, table, W, b):
    scale = math.sqrt(D_MODEL)
    idx3 = x.reshape(NW, R // ROWS_PER_STREAM, ROWS_PER_STREAM).astype(jnp.int32)
    emb2 = _sc_gather(idx3, table).reshape(N // 2, 2 * EMBED)
    Ws = W * scale
    zero = jnp.zeros_like(Ws)
    W2 = jnp.block([[Ws, zero], [zero, Ws]])
    b2 = jnp.tile(b * scale, 2).reshape(1, 2 * D_MODEL)
    out = _tc_matmul(emb2, W2, b2)
    return out.reshape(B, L, D_MODEL)


# speedup vs baseline: 1.5874x; 1.5874x over previous
import math
import functools

import jax
import jax.numpy as jnp
from jax import lax
from jax.experimental import pallas as pl
from jax.experimental.pallas import tpu as pltpu
from jax.experimental.pallas import tpu_sc as plsc

VOCAB = 1000000
EMBED = 64
D_MODEL = 128
B = 4096
L = 200

NC = 2
NS = 16
NW = NC * NS

N = B * L
R = N // NW
BLOCK = 12800
HALF = BLOCK // 2
NBLK = R // BLOCK
CHUNK = 640
NCHUNK = HALF // CHUNK
SPS = CHUNK // 128
NSTREAM = R // 128


def _sc_gather_packed(idx3, table):
    mesh = plsc.VectorSubcoreMesh(core_axis_name="c", subcore_axis_name="s")

    @functools.partial(
        pl.kernel,
        out_type=jax.ShapeDtypeStruct((N // 2, 2 * EMBED), jnp.float32),
        mesh=mesh,
        scratch_types=[
            pltpu.VMEM((NSTREAM, 128), jnp.int32),
            pltpu.VMEM((CHUNK, EMBED), jnp.float32),
            pltpu.SemaphoreType.DMA,
        ],
        compiler_params=pltpu.CompilerParams(use_tc_tiling_on_sc=False),
    )
    def k(idx_hbm, table_hbm, emb_hbm, idx_v, rows_v, sem):
        wid = lax.axis_index("s") * NC + lax.axis_index("c")
        row_base = wid * (R // 2)

        pltpu.sync_copy(idx_hbm.at[wid], idx_v)

        def chunk_body(t, carry):
            blk = t // (2 * NCHUNK)
            h = (t // NCHUNK) % 2
            c = t % NCHUNK
            descs = [
                pltpu.async_copy(
                    table_hbm.at[idx_v.at[t * SPS + q]],
                    rows_v.at[pl.ds(q * 128, 128)],
                    sem,
                )
                for q in range(SPS)
            ]
            for d in descs:
                d.wait()
            pltpu.sync_copy(
                rows_v,
                emb_hbm.at[
                    pl.ds(row_base + blk * HALF + c * CHUNK, CHUNK),
                    pl.ds(h * EMBED, EMBED),
                ],
            )
            return carry

        lax.fori_loop(0, NBLK * 2 * NCHUNK, chunk_body, 0)

    return k(idx3, table)


def _tc_matmul(emb2, Wa, Wb, b2):
    def body(emb_ref, wa_ref, wb_ref, b_ref, out_ref):
        e = emb_ref[...]
        top = jnp.dot(e, wa_ref[...], preferred_element_type=jnp.float32)
        bot = jnp.dot(e, wb_ref[...], preferred_element_type=jnp.float32)
        top = top + b_ref[...]
        bot = bot + b_ref[...]
        half_rows = BLOCK // L // 2
        out_ref[0:half_rows] = top.reshape(half_rows, L, D_MODEL)
        out_ref[half_rows:] = bot.reshape(half_rows, L, D_MODEL)

    return pl.pallas_call(
        body,
        grid=(N // BLOCK,),
        in_specs=[
            pl.BlockSpec((HALF, 2 * EMBED), lambda i: (i, 0)),
            pl.BlockSpec((2 * EMBED, D_MODEL), lambda i: (0, 0)),
            pl.BlockSpec((2 * EMBED, D_MODEL), lambda i: (0, 0)),
            pl.BlockSpec((1, D_MODEL), lambda i: (0, 0)),
        ],
        out_specs=pl.BlockSpec((BLOCK // L, L, D_MODEL), lambda i: (i, 0, 0)),
        out_shape=jax.ShapeDtypeStruct((B, L, D_MODEL), jnp.float32),
    )(emb2, Wa, Wb, b2)


def kernel(x, table, W, b):
    scale = math.sqrt(D_MODEL)
    idx3 = x.reshape(NW, NSTREAM, 128).astype(jnp.int32)
    emb2 = _sc_gather_packed(idx3, table)
    Ws = W * scale
    zero = jnp.zeros_like(Ws)
    Wa = jnp.concatenate([Ws, zero], axis=0)
    Wb = jnp.concatenate([zero, Ws], axis=0)
    b2 = (b * scale).reshape(1, D_MODEL)
    return _tc_matmul(emb2, Wa, Wb, b2)
